# Initial kernel scaffold; baseline (speedup 1.0000x reference)
import jax
import jax.numpy as jnp
from jax.experimental import pallas as pl


def kernel(inputs, Wg, w1, w2, w3):
    raise NotImplementedError("write your pallas kernel here")


# speedup vs baseline: 1.4529x; 1.4529x over previous
import functools

import jax
import jax.numpy as jnp
from jax.experimental import pallas as pl
from jax.experimental.pallas import tpu as pltpu
from jax.experimental.pallas import tpu_sc as plsc

E = 8
K = 2
D = 1024
F = 2048
T = 4096
A = K * T

B_BLK = 256
S = A + E * B_BLK
NB = S // B_BLK
F_BLK = 1024
NF = F // F_BLK

NW = 32
T_PER_W = T // NW
CH_D = 64
CH_C = 32

_FP32 = jnp.float32
_I32 = jnp.int32


def _router_body(x_ref, wg_ref, pos_ref, wts_ref, be_ref):
    x = x_ref[...]
    wg = wg_ref[...]
    logits = jax.lax.dot_general(
        x, wg, (((1,), (1,)), ((), ())), preferred_element_type=_FP32)

    col = jax.lax.broadcasted_iota(_I32, (T, E), 1)
    m1 = jnp.max(logits, axis=1, keepdims=True)
    e1 = jnp.min(jnp.where(logits == m1, col, E), axis=1, keepdims=True)
    masked = jnp.where(col == e1, -jnp.inf, logits)
    m2 = jnp.max(masked, axis=1, keepdims=True)
    e2 = jnp.min(jnp.where(masked == m2, col, E), axis=1, keepdims=True)

    t = jnp.exp(m2 - m1)
    w_hi = 1.0 / (1.0 + t)
    w_lo = 1.0 - w_hi

    e_all = jnp.concatenate([e1, e2], axis=0)
    colA = jax.lax.broadcasted_iota(_I32, (A, E), 1)
    onehot = (colA == e_all).astype(_FP32)
    csum = onehot
    shift = 1
    while shift < A:
        csum = csum + jnp.concatenate(
            [jnp.zeros((shift, E), _FP32), csum[: A - shift]], axis=0)
        shift *= 2
    counts = csum[A - 1:A, :]

    padded = jnp.ceil(counts / B_BLK) * B_BLK
    rowE = jax.lax.broadcasted_iota(_I32, (E, E), 0)
    colE = jax.lax.broadcasted_iota(_I32, (E, E), 1)
    strictly_lt = (rowE < colE).astype(_FP32)
    start = jax.lax.dot_general(
        padded, strictly_lt, (((1,), (0,)), ((), ())),
        preferred_element_type=_FP32)

    pos = jnp.sum((csum - onehot + start) * onehot, axis=1, keepdims=True)
    pos_ref[...] = pos.astype(_I32)
    wts_ref[...] = jnp.concatenate([w_hi, w_lo], axis=0)

    start_col = jax.lax.dot_general(
        (rowE == colE).astype(_FP32), start, (((1,), (1,)), ((), ())),
        preferred_element_type=_FP32)
    blk = jax.lax.broadcasted_iota(_I32, (E, 128), 1).astype(_FP32) * B_BLK
    sl = jax.lax.broadcasted_iota(_I32, (E, 128), 0)
    hit = jnp.where(sl >= 1, (blk >= start_col).astype(_I32), 0)
    be_ref[...] = jnp.sum(hit, axis=0, keepdims=True)


def _router(inputs, Wg):
    return pl.pallas_call(
        _router_body,
        out_shape=[
            jax.ShapeDtypeStruct((A, 1), _I32),
            jax.ShapeDtypeStruct((A, 1), _FP32),
            jax.ShapeDtypeStruct((1, 128), _I32),
        ],
    )(inputs, Wg)


def _dispatch_body(x_hbm, pos_hbm, xd_hbm, idx_v, rows_v):
    wid = jax.lax.axis_index("s") * 2 + jax.lax.axis_index("c")
    for j in range(K):
        @pl.loop(0, T_PER_W // CH_D)
        def _(it, j=j):
            base = wid * T_PER_W + it * CH_D
            pltpu.sync_copy(pos_hbm.at[j, pl.ds(base, CH_D)], idx_v)
            pltpu.sync_copy(x_hbm.at[pl.ds(base, CH_D)], rows_v)
            pltpu.sync_copy(rows_v, xd_hbm.at[idx_v])


@functools.lru_cache(maxsize=1)
def _sc_kernels():
    mesh = plsc.VectorSubcoreMesh(core_axis_name="c", subcore_axis_name="s")
    dispatch = functools.partial(
        pl.kernel,
        mesh=mesh,
        out_type=jax.ShapeDtypeStruct((S, D), _FP32),
        scratch_types=[
            pltpu.VMEM((CH_D,), _I32),
            pltpu.VMEM((CH_D, D), _FP32),
        ],
    )(_dispatch_body)
    combine = functools.partial(
        pl.kernel,
        mesh=mesh,
        out_type=jax.ShapeDtypeStruct((T, D), _FP32),
        compiler_params=pltpu.CompilerParams(needs_layout_passes=False),
        scratch_types=[
            pltpu.VMEM((CH_C,), _I32),
            pltpu.VMEM((CH_C,), _I32),
            pltpu.VMEM((CH_C,), _FP32),
            pltpu.VMEM((CH_C,), _FP32),
            pltpu.VMEM((CH_C, D), _FP32),
            pltpu.VMEM((CH_C, D), _FP32),
            pltpu.VMEM((CH_C, D), _FP32),
            pltpu.SemaphoreType.DMA,
            pltpu.SemaphoreType.DMA,
        ],
    )(_combine_body)
    return dispatch, combine


def _ffn_body(be_ref, x_ref, w1_ref, w3_ref, w2_ref, y_ref, acc_ref):
    f = pl.program_id(1)
    x = x_ref[...]
    a = jax.lax.dot_general(
        x, w1_ref[0], (((1,), (1,)), ((), ())), preferred_element_type=_FP32)
    c = jax.lax.dot_general(
        x, w3_ref[0], (((1,), (1,)), ((), ())), preferred_element_type=_FP32)
    h = a * jax.nn.sigmoid(a) * c
    contrib = jax.lax.dot_general(
        h, w2_ref[0], (((1,), (1,)), ((), ())), preferred_element_type=_FP32)

    @pl.when(f == 0)
    def _():
        acc_ref[...] = contrib

    @pl.when(f > 0)
    def _():
        acc_ref[...] += contrib

    @pl.when(f == NF - 1)
    def _():
        y_ref[...] = acc_ref[...]


def _ffn(be, xd, w1, w2, w3):
    grid_spec = pltpu.PrefetchScalarGridSpec(
        num_scalar_prefetch=1,
        grid=(NB, NF),
        in_specs=[
            pl.BlockSpec((B_BLK, D), lambda b, f, be: (b, 0)),
            pl.BlockSpec((1, F_BLK, D), lambda b, f, be: (be[b], f, 0)),
            pl.BlockSpec((1, F_BLK, D), lambda b, f, be: (be[b], f, 0)),
            pl.BlockSpec((1, D, F_BLK), lambda b, f, be: (be[b], 0, f)),
        ],
        out_specs=pl.BlockSpec((B_BLK, D), lambda b, f, be: (b, 0)),
        scratch_shapes=[pltpu.VMEM((B_BLK, D), _FP32)],
    )
    return pl.pallas_call(
        _ffn_body,
        grid_spec=grid_spec,
        out_shape=jax.ShapeDtypeStruct((S, D), _FP32),
        compiler_params=pltpu.CompilerParams(
            dimension_semantics=("arbitrary", "arbitrary")),
    )(be, xd, w1, w3, w2)


def _combine_body(y_hbm, pos_hbm, w_hbm, out_hbm,
                  idx0, idx1, w0v, w1v, g0, g1, ov, sem0, sem1):
    wid = jax.lax.axis_index("s") * 2 + jax.lax.axis_index("c")

    @pl.loop(0, T_PER_W // CH_C)
    def _(it):
        base = wid * T_PER_W + it * CH_C
        pltpu.sync_copy(pos_hbm.at[0, pl.ds(base, CH_C)], idx0)
        pltpu.sync_copy(pos_hbm.at[1, pl.ds(base, CH_C)], idx1)
        pltpu.sync_copy(w_hbm.at[0, pl.ds(base, CH_C)], w0v)
        pltpu.sync_copy(w_hbm.at[1, pl.ds(base, CH_C)], w1v)
        cp0 = pltpu.async_copy(y_hbm.at[idx0], g0, sem0)
        cp1 = pltpu.async_copy(y_hbm.at[idx1], g1, sem1)
        cp0.wait()
        cp1.wait()

        @pl.loop(0, CH_C)
        def _(r):
            lane = jnp.full((16,), r, _I32)
            w0s = plsc.load_gather(w0v, [lane])
            w1s = plsc.load_gather(w1v, [lane])

            @pl.loop(0, D, step=16)
            def _(c):
                ov[r, pl.ds(c, 16)] = (
                    g0[r, pl.ds(c, 16)] * w0s + g1[r, pl.ds(c, 16)] * w1s)

        pltpu.sync_copy(ov, out_hbm.at[pl.ds(base, CH_C)])


def kernel(inputs, Wg, w1, w2, w3):
    pos, wts, be_pad = _router(inputs, Wg)
    pos2 = pos.reshape(K, T)
    wts2 = wts.reshape(K, T)
    be = be_pad[0, :NB]
    dispatch, combine = _sc_kernels()
    xd = dispatch(inputs, pos2)
    yd = _ffn(be, xd, w1, w2, w3)
    return combine(yd, pos2, wts2)


# speedup vs baseline: 2.0272x; 1.3952x over previous
import functools

import jax
import jax.numpy as jnp
from jax.experimental import pallas as pl
from jax.experimental.pallas import tpu as pltpu
from jax.experimental.pallas import tpu_sc as plsc

E = 8
K = 2
D = 1024
F = 2048
T = 4096
A = K * T

B_BLK = 256
S = A + E * B_BLK
NB = S // B_BLK
F_BLK = 2048
NF = F // F_BLK

NW = 32
T_PER_W = T // NW
CH_D = 64
CH_C = 32

_FP32 = jnp.float32
_I32 = jnp.int32


def _router_body(x_ref, wg_ref, pos_ref, wts_ref, be_ref):
    x = x_ref[...]
    wg = wg_ref[...]
    logits = jax.lax.dot_general(
        x, wg, (((1,), (1,)), ((), ())), preferred_element_type=_FP32)

    col = jax.lax.broadcasted_iota(_I32, (T, E), 1)
    m1 = jnp.max(logits, axis=1, keepdims=True)
    e1 = jnp.min(jnp.where(logits == m1, col, E), axis=1, keepdims=True)
    masked = jnp.where(col == e1, -jnp.inf, logits)
    m2 = jnp.max(masked, axis=1, keepdims=True)
    e2 = jnp.min(jnp.where(masked == m2, col, E), axis=1, keepdims=True)

    t = jnp.exp(m2 - m1)
    w_hi = 1.0 / (1.0 + t)
    w_lo = 1.0 - w_hi

    e_all = jnp.concatenate([e1, e2], axis=0)
    colA = jax.lax.broadcasted_iota(_I32, (A, E), 1)
    onehot = (colA == e_all).astype(_FP32)
    csum = onehot
    shift = 1
    while shift < A:
        csum = csum + jnp.concatenate(
            [jnp.zeros((shift, E), _FP32), csum[: A - shift]], axis=0)
        shift *= 2
    counts = csum[A - 1:A, :]

    padded = jnp.ceil(counts / B_BLK) * B_BLK
    rowE = jax.lax.broadcasted_iota(_I32, (E, E), 0)
    colE = jax.lax.broadcasted_iota(_I32, (E, E), 1)
    strictly_lt = (rowE < colE).astype(_FP32)
    start = jax.lax.dot_general(
        padded, strictly_lt, (((1,), (0,)), ((), ())),
        preferred_element_type=_FP32)

    pos = jnp.sum((csum - onehot + start) * onehot, axis=1, keepdims=True)
    pos_ref[...] = pos.astype(_I32)
    wts_ref[...] = jnp.concatenate([w_hi, w_lo], axis=0)

    start_col = jax.lax.dot_general(
        (rowE == colE).astype(_FP32), start, (((1,), (1,)), ((), ())),
        preferred_element_type=_FP32)
    blk = jax.lax.broadcasted_iota(_I32, (E, 128), 1).astype(_FP32) * B_BLK
    sl = jax.lax.broadcasted_iota(_I32, (E, 128), 0)
    hit = jnp.where(sl >= 1, (blk >= start_col).astype(_I32), 0)
    be_ref[...] = jnp.sum(hit, axis=0, keepdims=True)


def _router(inputs, Wg):
    return pl.pallas_call(
        _router_body,
        out_shape=[
            jax.ShapeDtypeStruct((A, 1), _I32),
            jax.ShapeDtypeStruct((A, 1), _FP32),
            jax.ShapeDtypeStruct((1, 128), _I32),
        ],
    )(inputs, Wg)


def _dispatch_body(x_hbm, pos_hbm, xd_hbm, idx_v, rows_v):
    wid = jax.lax.axis_index("s") * 2 + jax.lax.axis_index("c")
    for j in range(K):
        @pl.loop(0, T_PER_W // CH_D)
        def _(it, j=j):
            base = wid * T_PER_W + it * CH_D
            pltpu.sync_copy(pos_hbm.at[j, pl.ds(base, CH_D)], idx_v)
            pltpu.sync_copy(x_hbm.at[pl.ds(base, CH_D)], rows_v)
            pltpu.sync_copy(rows_v, xd_hbm.at[idx_v])


@functools.lru_cache(maxsize=1)
def _sc_kernels():
    mesh = plsc.VectorSubcoreMesh(core_axis_name="c", subcore_axis_name="s")
    dispatch = functools.partial(
        pl.kernel,
        mesh=mesh,
        out_type=jax.ShapeDtypeStruct((S, D), _FP32),
        scratch_types=[
            pltpu.VMEM((CH_D,), _I32),
            pltpu.VMEM((CH_D, D), _FP32),
        ],
    )(_dispatch_body)
    combine = functools.partial(
        pl.kernel,
        mesh=mesh,
        out_type=jax.ShapeDtypeStruct((T, D), _FP32),
        compiler_params=pltpu.CompilerParams(needs_layout_passes=False),
        scratch_types=[
            pltpu.VMEM((CH_C,), _I32),
            pltpu.VMEM((CH_C,), _I32),
            pltpu.VMEM((CH_C,), _FP32),
            pltpu.VMEM((CH_C,), _FP32),
            pltpu.VMEM((CH_C, D), _FP32),
            pltpu.VMEM((CH_C, D), _FP32),
            pltpu.VMEM((CH_C, D), _FP32),
            pltpu.SemaphoreType.DMA,
            pltpu.SemaphoreType.DMA,
        ],
    )(_combine_body)
    return dispatch, combine


def _ffn_body(be_ref, x_ref, w1_ref, w3_ref, w2_ref, y_ref):
    x = x_ref[...]
    a = jax.lax.dot_general(
        x, w1_ref[0], (((1,), (1,)), ((), ())), preferred_element_type=_FP32)
    c = jax.lax.dot_general(
        x, w3_ref[0], (((1,), (1,)), ((), ())), preferred_element_type=_FP32)
    h = a * jax.nn.sigmoid(a) * c
    y_ref[...] = jax.lax.dot_general(
        h, w2_ref[0], (((1,), (1,)), ((), ())), preferred_element_type=_FP32)


def _ffn(be, xd, w1, w2, w3):
    grid_spec = pltpu.PrefetchScalarGridSpec(
        num_scalar_prefetch=1,
        grid=(NB,),
        in_specs=[
            pl.BlockSpec((B_BLK, D), lambda b, be: (b, 0)),
            pl.BlockSpec((1, F_BLK, D), lambda b, be: (be[b], 0, 0)),
            pl.BlockSpec((1, F_BLK, D), lambda b, be: (be[b], 0, 0)),
            pl.BlockSpec((1, D, F_BLK), lambda b, be: (be[b], 0, 0)),
        ],
        out_specs=pl.BlockSpec((B_BLK, D), lambda b, be: (b, 0)),
    )
    return pl.pallas_call(
        _ffn_body,
        grid_spec=grid_spec,
        out_shape=jax.ShapeDtypeStruct((S, D), _FP32),
        compiler_params=pltpu.CompilerParams(
            dimension_semantics=("arbitrary",)),
    )(be, xd, w1, w3, w2)


def _combine_body(y_hbm, pos_hbm, w_hbm, out_hbm,
                  idx0, idx1, w0v, w1v, g0, g1, ov, sem0, sem1):
    wid = jax.lax.axis_index("s") * 2 + jax.lax.axis_index("c")

    @pl.loop(0, T_PER_W // CH_C)
    def _(it):
        base = wid * T_PER_W + it * CH_C
        pltpu.sync_copy(pos_hbm.at[0, pl.ds(base, CH_C)], idx0)
        pltpu.sync_copy(pos_hbm.at[1, pl.ds(base, CH_C)], idx1)
        pltpu.sync_copy(w_hbm.at[0, pl.ds(base, CH_C)], w0v)
        pltpu.sync_copy(w_hbm.at[1, pl.ds(base, CH_C)], w1v)
        cp0 = pltpu.async_copy(y_hbm.at[idx0], g0, sem0)
        cp1 = pltpu.async_copy(y_hbm.at[idx1], g1, sem1)
        cp0.wait()
        cp1.wait()

        @pl.loop(0, CH_C)
        def _(r):
            lane = jnp.full((16,), r, _I32)
            w0s = plsc.load_gather(w0v, [lane])
            w1s = plsc.load_gather(w1v, [lane])

            @pl.loop(0, D, step=16)
            def _(c):
                ov[r, pl.ds(c, 16)] = (
                    g0[r, pl.ds(c, 16)] * w0s + g1[r, pl.ds(c, 16)] * w1s)

        pltpu.sync_copy(ov, out_hbm.at[pl.ds(base, CH_C)])


def kernel(inputs, Wg, w1, w2, w3):
    pos, wts, be_pad = _router(inputs, Wg)
    pos2 = pos.reshape(K, T)
    wts2 = wts.reshape(K, T)
    be = be_pad[0, :NB]
    dispatch, combine = _sc_kernels()
    xd = dispatch(inputs, pos2)
    yd = _ffn(be, xd, w1, w2, w3)
    return combine(yd, pos2, wts2)


# speedup vs baseline: 2.0745x; 1.0234x over previous
import functools

import jax
import jax.numpy as jnp
from jax.experimental import pallas as pl
from jax.experimental.pallas import tpu as pltpu
from jax.experimental.pallas import tpu_sc as plsc

E = 8
K = 2
D = 1024
F = 2048
T = 4096
A = K * T

B_BLK = 256
S = A + E * B_BLK
NB = S // B_BLK
F_BLK = 2048
NF = F // F_BLK

NW = 32
T_PER_W = T // NW
CH_D = 64
CH_C = 32

_FP32 = jnp.float32
_I32 = jnp.int32


def _router_body(x_ref, wg_ref, pos_ref, wts_ref, be_ref):
    x = x_ref[...]
    wg = wg_ref[...]
    logits = jax.lax.dot_general(
        x, wg, (((1,), (1,)), ((), ())), preferred_element_type=_FP32)

    col = jax.lax.broadcasted_iota(_I32, (T, E), 1)
    m1 = jnp.max(logits, axis=1, keepdims=True)
    e1 = jnp.min(jnp.where(logits == m1, col, E), axis=1, keepdims=True)
    masked = jnp.where(col == e1, -jnp.inf, logits)
    m2 = jnp.max(masked, axis=1, keepdims=True)
    e2 = jnp.min(jnp.where(masked == m2, col, E), axis=1, keepdims=True)

    t = jnp.exp(m2 - m1)
    w_hi = 1.0 / (1.0 + t)
    w_lo = 1.0 - w_hi

    e_all = jnp.concatenate([e1, e2], axis=0)
    colA = jax.lax.broadcasted_iota(_I32, (A, E), 1)
    onehot = (colA == e_all).astype(_FP32)
    csum = onehot
    shift = 1
    while shift < A:
        csum = csum + jnp.concatenate(
            [jnp.zeros((shift, E), _FP32), csum[: A - shift]], axis=0)
        shift *= 2
    counts = csum[A - 1:A, :]

    padded = jnp.ceil(counts / B_BLK) * B_BLK
    rowE = jax.lax.broadcasted_iota(_I32, (E, E), 0)
    colE = jax.lax.broadcasted_iota(_I32, (E, E), 1)
    strictly_lt = (rowE < colE).astype(_FP32)
    start = jax.lax.dot_general(
        padded, strictly_lt, (((1,), (0,)), ((), ())),
        preferred_element_type=_FP32)

    pos = jnp.sum((csum - onehot + start) * onehot, axis=1, keepdims=True)
    pos_ref[...] = pos.astype(_I32)
    wts_ref[...] = jnp.concatenate([w_hi, w_lo], axis=0)

    start_col = jax.lax.dot_general(
        (rowE == colE).astype(_FP32), start, (((1,), (1,)), ((), ())),
        preferred_element_type=_FP32)
    blk = jax.lax.broadcasted_iota(_I32, (E, 128), 1).astype(_FP32) * B_BLK
    sl = jax.lax.broadcasted_iota(_I32, (E, 128), 0)
    hit = jnp.where(sl >= 1, (blk >= start_col).astype(_I32), 0)
    be_ref[...] = jnp.sum(hit, axis=0, keepdims=True)


def _router(inputs, Wg):
    return pl.pallas_call(
        _router_body,
        out_shape=[
            jax.ShapeDtypeStruct((A, 1), _I32),
            jax.ShapeDtypeStruct((A, 1), _FP32),
            jax.ShapeDtypeStruct((1, 128), _I32),
        ],
    )(inputs, Wg)


def _dispatch_body(x_hbm, pos_hbm, xd_hbm, idx_v, rows_v):
    wid = jax.lax.axis_index("s") * 2 + jax.lax.axis_index("c")

    @pl.loop(0, T_PER_W // CH_D)
    def _(it):
        base = wid * T_PER_W + it * CH_D
        pltpu.sync_copy(x_hbm.at[pl.ds(base, CH_D)], rows_v)
        for j in range(K):
            pltpu.sync_copy(pos_hbm.at[j, pl.ds(base, CH_D)], idx_v)
            pltpu.sync_copy(rows_v, xd_hbm.at[idx_v])


@functools.lru_cache(maxsize=1)
def _sc_kernels():
    mesh = plsc.VectorSubcoreMesh(core_axis_name="c", subcore_axis_name="s")
    dispatch = functools.partial(
        pl.kernel,
        mesh=mesh,
        out_type=jax.ShapeDtypeStruct((S, D), _FP32),
        scratch_types=[
            pltpu.VMEM((CH_D,), _I32),
            pltpu.VMEM((CH_D, D), _FP32),
        ],
    )(_dispatch_body)
    combine = functools.partial(
        pl.kernel,
        mesh=mesh,
        out_type=jax.ShapeDtypeStruct((T, D), _FP32),
        compiler_params=pltpu.CompilerParams(needs_layout_passes=False),
        scratch_types=[
            pltpu.VMEM((CH_C,), _I32),
            pltpu.VMEM((CH_C,), _I32),
            pltpu.VMEM((CH_C,), _FP32),
            pltpu.VMEM((CH_C,), _FP32),
            pltpu.VMEM((CH_C, D), _FP32),
            pltpu.VMEM((CH_C, D), _FP32),
            pltpu.VMEM((CH_C, D), _FP32),
            pltpu.SemaphoreType.DMA,
            pltpu.SemaphoreType.DMA,
        ],
    )(_combine_body)
    return dispatch, combine


def _ffn_body(be_ref, x_ref, w1_ref, w3_ref, w2_ref, y_ref):
    x = x_ref[...]
    a = jax.lax.dot_general(
        x, w1_ref[0], (((1,), (1,)), ((), ())), preferred_element_type=_FP32)
    c = jax.lax.dot_general(
        x, w3_ref[0], (((1,), (1,)), ((), ())), preferred_element_type=_FP32)
    h = (a * (0.5 * jnp.tanh(0.5 * a) + 0.5)) * c
    y_ref[...] = jax.lax.dot_general(
        h, w2_ref[0], (((1,), (1,)), ((), ())), preferred_element_type=_FP32)


def _ffn(be, xd, w1, w2, w3):
    grid_spec = pltpu.PrefetchScalarGridSpec(
        num_scalar_prefetch=1,
        grid=(NB,),
        in_specs=[
            pl.BlockSpec((B_BLK, D), lambda b, be: (b, 0)),
            pl.BlockSpec((1, F_BLK, D), lambda b, be: (be[b], 0, 0)),
            pl.BlockSpec((1, F_BLK, D), lambda b, be: (be[b], 0, 0)),
            pl.BlockSpec((1, D, F_BLK), lambda b, be: (be[b], 0, 0)),
        ],
        out_specs=pl.BlockSpec((B_BLK, D), lambda b, be: (b, 0)),
    )
    return pl.pallas_call(
        _ffn_body,
        grid_spec=grid_spec,
        out_shape=jax.ShapeDtypeStruct((S, D), _FP32),
        compiler_params=pltpu.CompilerParams(
            dimension_semantics=("arbitrary",)),
    )(be, xd, w1, w3, w2)


def _combine_body(y_hbm, pos_hbm, w_hbm, out_hbm,
                  idx0, idx1, w0v, w1v, g0, g1, ov, sem0, sem1):
    wid = jax.lax.axis_index("s") * 2 + jax.lax.axis_index("c")

    @pl.loop(0, T_PER_W // CH_C)
    def _(it):
        base = wid * T_PER_W + it * CH_C
        pltpu.sync_copy(pos_hbm.at[0, pl.ds(base, CH_C)], idx0)
        pltpu.sync_copy(pos_hbm.at[1, pl.ds(base, CH_C)], idx1)
        pltpu.sync_copy(w_hbm.at[0, pl.ds(base, CH_C)], w0v)
        pltpu.sync_copy(w_hbm.at[1, pl.ds(base, CH_C)], w1v)
        cp0 = pltpu.async_copy(y_hbm.at[idx0], g0, sem0)
        cp1 = pltpu.async_copy(y_hbm.at[idx1], g1, sem1)
        cp0.wait()
        cp1.wait()

        @pl.loop(0, CH_C)
        def _(r):
            lane = jnp.full((16,), r, _I32)
            w0s = plsc.load_gather(w0v, [lane])
            w1s = plsc.load_gather(w1v, [lane])

            @pl.loop(0, D, step=16)
            def _(c):
                ov[r, pl.ds(c, 16)] = (
                    g0[r, pl.ds(c, 16)] * w0s + g1[r, pl.ds(c, 16)] * w1s)

        pltpu.sync_copy(ov, out_hbm.at[pl.ds(base, CH_C)])


def kernel(inputs, Wg, w1, w2, w3):
    pos, wts, be_pad = _router(inputs, Wg)
    pos2 = pos.reshape(K, T)
    wts2 = wts.reshape(K, T)
    be = be_pad[0, :NB]
    dispatch, combine = _sc_kernels()
    xd = dispatch(inputs, pos2)
    yd = _ffn(be, xd, w1, w2, w3)
    return combine(yd, pos2, wts2)


# speedup vs baseline: 2.1550x; 1.0388x over previous
import functools

import jax
import jax.numpy as jnp
from jax.experimental import pallas as pl
from jax.experimental.pallas import tpu as pltpu
from jax.experimental.pallas import tpu_sc as plsc

E = 8
K = 2
D = 1024
F = 2048
T = 4096
A = K * T

B_BLK = 256
S = A + E * B_BLK
NB = S // B_BLK
F_BLK = 2048
NF = F // F_BLK

NW = 32
T_PER_W = T // NW
CH_D = 64
CH_C = 32

_FP32 = jnp.float32
_I32 = jnp.int32


def _router_body(x_ref, wg_ref, pos_ref, wts_ref, be_ref):
    x = x_ref[...]
    wg = wg_ref[...]
    lt = jax.lax.dot_general(
        wg, x, (((1,), (1,)), ((), ())), preferred_element_type=_FP32)

    row = jax.lax.broadcasted_iota(_I32, (E, T), 0)
    m1 = jnp.max(lt, axis=0, keepdims=True)
    e1 = jnp.min(jnp.where(lt == m1, row, E), axis=0, keepdims=True)
    masked = jnp.where(row == e1, -jnp.inf, lt)
    m2 = jnp.max(masked, axis=0, keepdims=True)
    e2 = jnp.min(jnp.where(masked == m2, row, E), axis=0, keepdims=True)

    t = jnp.exp(m2 - m1)
    w_hi = 1.0 / (1.0 + t)
    w_lo = 1.0 - w_hi

    R, L = A // 128, 128
    e_lay = jnp.concatenate([e1, e2], axis=0).reshape(R, L)
    wts_ref[...] = jnp.concatenate([w_hi, w_lo], axis=0).reshape(R, L)

    li = jax.lax.broadcasted_iota(_I32, (L, L), 0)
    lj = jax.lax.broadcasted_iota(_I32, (L, L), 1)
    U = (li <= lj).astype(_FP32)
    ri = jax.lax.broadcasted_iota(_I32, (R, R), 0)
    rj = jax.lax.broadcasted_iota(_I32, (R, R), 1)
    Lo = (rj < ri).astype(_FP32)

    blk = jax.lax.broadcasted_iota(_I32, (1, 128), 1).astype(_FP32) * B_BLK
    pos_acc = jnp.zeros((R, L), _FP32)
    be_acc = jnp.zeros((1, 128), _I32)
    start = jnp.zeros((1, 1), _FP32)
    for e in range(E):
        m = (e_lay == e).astype(_FP32)
        p = jax.lax.dot_general(
            m, U, (((1,), (0,)), ((), ())), preferred_element_type=_FP32)
        tot = p[:, L - 1 : L]
        carry = jax.lax.dot_general(
            Lo, tot, (((1,), (0,)), ((), ())), preferred_element_type=_FP32)
        count = carry[R - 1 : R, :] + tot[R - 1 : R, :]
        pos_e = p - 1.0 + carry + start
        pos_acc = pos_acc + m * pos_e
        if e >= 1:
            be_acc = be_acc + (blk >= start).astype(_I32)
        start = start + jnp.ceil(count / B_BLK) * B_BLK
    pos_ref[...] = pos_acc.astype(_I32)
    be_ref[...] = be_acc


def _router(inputs, Wg):
    return pl.pallas_call(
        _router_body,
        out_shape=[
            jax.ShapeDtypeStruct((A // 128, 128), _I32),
            jax.ShapeDtypeStruct((A // 128, 128), _FP32),
            jax.ShapeDtypeStruct((1, 128), _I32),
        ],
    )(inputs, Wg)


def _dispatch_body(x_hbm, pos_hbm, xd_hbm, idx_v, rows_v):
    wid = jax.lax.axis_index("s") * 2 + jax.lax.axis_index("c")

    @pl.loop(0, T_PER_W // CH_D)
    def _(it):
        base = wid * T_PER_W + it * CH_D
        pltpu.sync_copy(x_hbm.at[pl.ds(base, CH_D)], rows_v)
        for j in range(K):
            pltpu.sync_copy(pos_hbm.at[pl.ds(j * T + base, CH_D)], idx_v)
            pltpu.sync_copy(rows_v, xd_hbm.at[idx_v])


@functools.lru_cache(maxsize=1)
def _sc_kernels():
    mesh = plsc.VectorSubcoreMesh(core_axis_name="c", subcore_axis_name="s")
    dispatch = functools.partial(
        pl.kernel,
        mesh=mesh,
        out_type=jax.ShapeDtypeStruct((S, D), _FP32),
        scratch_types=[
            pltpu.VMEM((CH_D,), _I32),
            pltpu.VMEM((CH_D, D), _FP32),
        ],
    )(_dispatch_body)
    combine = functools.partial(
        pl.kernel,
        mesh=mesh,
        out_type=jax.ShapeDtypeStruct((T, D), _FP32),
        compiler_params=pltpu.CompilerParams(needs_layout_passes=False),
        scratch_types=[
            pltpu.VMEM((CH_C,), _I32),
            pltpu.VMEM((CH_C,), _I32),
            pltpu.VMEM((CH_C,), _FP32),
            pltpu.VMEM((CH_C,), _FP32),
            pltpu.VMEM((CH_C, D), _FP32),
            pltpu.VMEM((CH_C, D), _FP32),
            pltpu.VMEM((CH_C, D), _FP32),
            pltpu.SemaphoreType.DMA,
            pltpu.SemaphoreType.DMA,
        ],
    )(_combine_body)
    return dispatch, combine


def _ffn_body(be_ref, x_ref, w1_ref, w3_ref, w2_ref, y_ref):
    x = x_ref[...]
    a = jax.lax.dot_general(
        x, w1_ref[0], (((1,), (1,)), ((), ())), preferred_element_type=_FP32)
    c = jax.lax.dot_general(
        x, w3_ref[0], (((1,), (1,)), ((), ())), preferred_element_type=_FP32)
    h = (a * (0.5 * jnp.tanh(0.5 * a) + 0.5)) * c
    y_ref[...] = jax.lax.dot_general(
        h, w2_ref[0], (((1,), (1,)), ((), ())), preferred_element_type=_FP32)


def _ffn(be, xd, w1, w2, w3):
    grid_spec = pltpu.PrefetchScalarGridSpec(
        num_scalar_prefetch=1,
        grid=(NB,),
        in_specs=[
            pl.BlockSpec((B_BLK, D), lambda b, be: (b, 0)),
            pl.BlockSpec((1, F_BLK, D), lambda b, be: (be[b], 0, 0)),
            pl.BlockSpec((1, F_BLK, D), lambda b, be: (be[b], 0, 0)),
            pl.BlockSpec((1, D, F_BLK), lambda b, be: (be[b], 0, 0)),
        ],
        out_specs=pl.BlockSpec((B_BLK, D), lambda b, be: (b, 0)),
    )
    return pl.pallas_call(
        _ffn_body,
        grid_spec=grid_spec,
        out_shape=jax.ShapeDtypeStruct((S, D), _FP32),
        compiler_params=pltpu.CompilerParams(
            dimension_semantics=("arbitrary",)),
    )(be, xd, w1, w3, w2)


def _combine_body(y_hbm, pos_hbm, w_hbm, out_hbm,
                  idx0, idx1, w0v, w1v, g0, g1, ov, sem0, sem1):
    wid = jax.lax.axis_index("s") * 2 + jax.lax.axis_index("c")

    @pl.loop(0, T_PER_W // CH_C)
    def _(it):
        base = wid * T_PER_W + it * CH_C
        pltpu.sync_copy(pos_hbm.at[pl.ds(base, CH_C)], idx0)
        pltpu.sync_copy(pos_hbm.at[pl.ds(T + base, CH_C)], idx1)
        pltpu.sync_copy(w_hbm.at[pl.ds(base, CH_C)], w0v)
        pltpu.sync_copy(w_hbm.at[pl.ds(T + base, CH_C)], w1v)
        cp0 = pltpu.async_copy(y_hbm.at[idx0], g0, sem0)
        cp1 = pltpu.async_copy(y_hbm.at[idx1], g1, sem1)
        cp0.wait()
        cp1.wait()

        @pl.loop(0, CH_C)
        def _(r):
            lane = jnp.full((16,), r, _I32)
            w0s = plsc.load_gather(w0v, [lane])
            w1s = plsc.load_gather(w1v, [lane])

            @pl.loop(0, D, step=16)
            def _(c):
                ov[r, pl.ds(c, 16)] = (
                    g0[r, pl.ds(c, 16)] * w0s + g1[r, pl.ds(c, 16)] * w1s)

        pltpu.sync_copy(ov, out_hbm.at[pl.ds(base, CH_C)])


def kernel(inputs, Wg, w1, w2, w3):
    pos, wts, be_pad = _router(inputs, Wg)
    pos1 = pos.reshape(A)
    wts1 = wts.reshape(A)
    be = be_pad[0, :NB]
    dispatch, combine = _sc_kernels()
    xd = dispatch(inputs, pos1)
    yd = _ffn(be, xd, w1, w2, w3)
    return combine(yd, pos1, wts1)
